# baseline (speedup 1.0000x reference)
import functools

import jax
import jax.numpy as jnp
from jax import lax
from jax.experimental import pallas as pl
from jax.experimental.pallas import tpu as pltpu
from jax.experimental.pallas import tpu_sc as plsc

N_NODES = 10000
N_EDGES = 320000
D = 128

NUM_CORES = 2
NUM_SUBCORES = 16
CHUNK = 128
ROWS_PER_SUB = 640
N_PAD = NUM_SUBCORES * ROWS_PER_SUB
IDX_BLOCK = 32
IDX_BLOCKS = 5
CHUNKS_PER_SUB = IDX_BLOCK * IDX_BLOCKS
EDGES_PER_SUB = CHUNKS_PER_SUB * CHUNK
E_PAD = NUM_SUBCORES * EDGES_PER_SUB


def _sc_mesh():
    return plsc.VectorSubcoreMesh(
        core_axis_name="c", subcore_axis_name="s",
        num_cores=NUM_CORES, num_subcores=NUM_SUBCORES)


@jax.jit
def _sc_degree(dst_all, ones_hbm, zeros_hbm):
    @functools.partial(
        pl.kernel,
        out_type=jax.ShapeDtypeStruct((NUM_CORES, N_PAD, D), jnp.float32),
        mesh=_sc_mesh(),
        scratch_types=[
            pltpu.VMEM((IDX_BLOCK, CHUNK), jnp.int32),
            pltpu.VMEM((CHUNK, D), jnp.float32),
            pltpu.VMEM_SHARED((N_PAD, D), jnp.float32),
            pltpu.SemaphoreType.DMA,
        ],
    )
    def hist_kernel(dst_hbm, ones_h, zeros_h, hist_hbm, idx_v, ones_v, acc_sh, sem):
        cid = lax.axis_index("c")
        sid = lax.axis_index("s")
        for t in range(ROWS_PER_SUB // CHUNK):
            pltpu.sync_copy(
                zeros_h, acc_sh.at[pl.ds(sid * ROWS_PER_SUB + t * CHUNK, CHUNK)])
        pltpu.sync_copy(ones_h, ones_v)
        plsc.subcore_barrier()

        dg = dst_hbm.at[cid].at[sid]

        @pl.loop(0, IDX_BLOCKS)
        def _(blk):
            pltpu.sync_copy(dg.at[pl.ds(blk * IDX_BLOCK, IDX_BLOCK)], idx_v)

            @pl.loop(0, IDX_BLOCK // 2)
            def _(j):
                a0 = pltpu.async_copy(ones_v, acc_sh.at[idx_v.at[2 * j]], sem,
                                      add=True)
                a1 = pltpu.async_copy(ones_v, acc_sh.at[idx_v.at[2 * j + 1]],
                                      sem, add=True)
                a0.wait()
                a1.wait()

        plsc.subcore_barrier()
        pltpu.sync_copy(
            acc_sh.at[pl.ds(sid * ROWS_PER_SUB, ROWS_PER_SUB)],
            hist_hbm.at[cid].at[pl.ds(sid * ROWS_PER_SUB, ROWS_PER_SUB)],
        )

    return hist_kernel(dst_all, ones_hbm, zeros_hbm)


@jax.jit
def _sc_scatter_add(z_all, src_all, dst_all, zrows_hbm):
    @functools.partial(
        pl.kernel,
        out_type=jax.ShapeDtypeStruct((NUM_CORES, N_PAD, D), jnp.float32),
        mesh=_sc_mesh(),
        scratch_types=[
            pltpu.VMEM((IDX_BLOCK, CHUNK), jnp.int32),
            pltpu.VMEM((IDX_BLOCK, CHUNK), jnp.int32),
            pltpu.VMEM((CHUNK, D), jnp.float32),
            pltpu.VMEM((CHUNK, D), jnp.float32),
            pltpu.VMEM_SHARED((N_PAD, D), jnp.float32),
            pltpu.SemaphoreType.DMA,
            pltpu.SemaphoreType.DMA,
            pltpu.SemaphoreType.DMA,
            pltpu.SemaphoreType.DMA,
        ],
    )
    def scat_kernel(z_hbm, src_hbm, dst_hbm, zr_hbm, agg_hbm,
                    src_v, dst_v, rows0, rows1, acc_sh,
                    semg0, semg1, sema0, sema1):
        cid = lax.axis_index("c")
        sid = lax.axis_index("s")
        for t in range(ROWS_PER_SUB // CHUNK):
            pltpu.sync_copy(
                zr_hbm, acc_sh.at[pl.ds(sid * ROWS_PER_SUB + t * CHUNK, CHUNK)])
        plsc.subcore_barrier()

        zg = z_hbm.at[cid]
        sg = src_hbm.at[cid].at[sid]
        dg = dst_hbm.at[cid].at[sid]

        @pl.loop(0, IDX_BLOCKS)
        def _(blk):
            pltpu.sync_copy(sg.at[pl.ds(blk * IDX_BLOCK, IDX_BLOCK)], src_v)
            pltpu.sync_copy(dg.at[pl.ds(blk * IDX_BLOCK, IDX_BLOCK)], dst_v)

            @pl.loop(0, IDX_BLOCK // 2)
            def _(p):
                c0 = 2 * p
                c1 = c0 + 1
                @pl.when(p > 0)
                def _():
                    pltpu.make_async_copy(
                        rows0, acc_sh.at[dst_v.at[c0 - 2]], sema0).wait()
                pltpu.async_copy(zg.at[src_v.at[c0]], rows0, semg0)

                @pl.when(p > 0)
                def _():
                    pltpu.make_async_copy(
                        rows1, acc_sh.at[dst_v.at[c0 - 1]], sema1).wait()
                pltpu.async_copy(zg.at[src_v.at[c1]], rows1, semg1)

                pltpu.make_async_copy(zg.at[src_v.at[c0]], rows0, semg0).wait()
                pltpu.async_copy(rows0, acc_sh.at[dst_v.at[c0]], sema0,
                                 add=True)
                pltpu.make_async_copy(zg.at[src_v.at[c1]], rows1, semg1).wait()
                pltpu.async_copy(rows1, acc_sh.at[dst_v.at[c1]], sema1,
                                 add=True)

            pltpu.make_async_copy(
                rows0, acc_sh.at[dst_v.at[IDX_BLOCK - 2]], sema0).wait()
            pltpu.make_async_copy(
                rows1, acc_sh.at[dst_v.at[IDX_BLOCK - 1]], sema1).wait()

        plsc.subcore_barrier()
        pltpu.sync_copy(
            acc_sh.at[pl.ds(sid * ROWS_PER_SUB, ROWS_PER_SUB)],
            agg_hbm.at[cid].at[pl.ds(sid * ROWS_PER_SUB, ROWS_PER_SUB)],
        )

    return scat_kernel(z_all, src_all, dst_all, zrows_hbm)


_ROWS = 2 * N_PAD
_BLK = 1024
_GRID = _ROWS // _BLK


def _row_mask(i):
    r = i * _BLK + lax.broadcasted_iota(jnp.int32, (_BLK, 1), 0)
    return (lax.rem(r, N_PAD) < N_NODES).astype(jnp.float32)


def _tc_matmul_body(x_ref, w_ref, xw_ref):
    xw_ref[...] = jnp.dot(x_ref[...], w_ref[...],
                          preferred_element_type=jnp.float32)


@jax.jit
def _tc_matmul(x_flat, w1):
    return pl.pallas_call(
        _tc_matmul_body,
        grid=(_GRID,),
        in_specs=[
            pl.BlockSpec((_BLK, D), lambda i: (i, 0)),
            pl.BlockSpec((D, D), lambda i: (0, 0)),
        ],
        out_specs=pl.BlockSpec((_BLK, D), lambda i: (i, 0)),
        out_shape=jax.ShapeDtypeStruct((_ROWS, D), jnp.float32),
    )(x_flat, w1)


def _tc_layer1_body(hist_ref, xw_ref, z_ref, dinv_ref):
    i = pl.program_id(0)
    deg = hist_ref[:, 0:1] + 1.0
    dinv = lax.rsqrt(deg) * _row_mask(i)
    z_ref[...] = dinv * xw_ref[...]
    dinv_ref[...] = jnp.broadcast_to(dinv, (_BLK, D))


@jax.jit
def _tc_layer1(hist_flat, xw_flat):
    return pl.pallas_call(
        _tc_layer1_body,
        grid=(_GRID,),
        in_specs=[
            pl.BlockSpec((_BLK, D), lambda i: (i, 0)),
            pl.BlockSpec((_BLK, D), lambda i: (i, 0)),
        ],
        out_specs=[
            pl.BlockSpec((_BLK, D), lambda i: (i, 0)),
            pl.BlockSpec((_BLK, D), lambda i: (i, 0)),
        ],
        out_shape=[
            jax.ShapeDtypeStruct((_ROWS, D), jnp.float32),
            jax.ShapeDtypeStruct((_ROWS, D), jnp.float32),
        ],
    )(hist_flat, xw_flat)


def _tc_layer2_body(agg_ref, z_ref, dinv_ref, b_ref, w_ref, z2_ref):
    dinv = dinv_ref[...]
    h = jnp.maximum(dinv * (agg_ref[...] + z_ref[...]) + b_ref[...], 0.0)
    z2_ref[...] = dinv * jnp.dot(h, w_ref[...],
                                 preferred_element_type=jnp.float32)


@jax.jit
def _tc_layer2(agg_flat, z_flat, dinv_flat, b1, w2):
    return pl.pallas_call(
        _tc_layer2_body,
        grid=(_GRID,),
        in_specs=[
            pl.BlockSpec((_BLK, D), lambda i: (i, 0)),
            pl.BlockSpec((_BLK, D), lambda i: (i, 0)),
            pl.BlockSpec((_BLK, D), lambda i: (i, 0)),
            pl.BlockSpec((1, D), lambda i: (0, 0)),
            pl.BlockSpec((D, D), lambda i: (0, 0)),
        ],
        out_specs=pl.BlockSpec((_BLK, D), lambda i: (i, 0)),
        out_shape=jax.ShapeDtypeStruct((_ROWS, D), jnp.float32),
    )(agg_flat, z_flat, dinv_flat, b1, w2)


def _tc_final_body(agg_ref, z_ref, dinv_ref, b_ref, out_ref):
    out_ref[...] = dinv_ref[...] * (agg_ref[...] + z_ref[...]) + b_ref[...]


@jax.jit
def _tc_final(agg_flat, z_flat, dinv_flat, b2):
    return pl.pallas_call(
        _tc_final_body,
        grid=(_GRID,),
        in_specs=[
            pl.BlockSpec((_BLK, D), lambda i: (i, 0)),
            pl.BlockSpec((_BLK, D), lambda i: (i, 0)),
            pl.BlockSpec((_BLK, D), lambda i: (i, 0)),
            pl.BlockSpec((1, D), lambda i: (0, 0)),
        ],
        out_specs=pl.BlockSpec((_BLK, D), lambda i: (i, 0)),
        out_shape=jax.ShapeDtypeStruct((_ROWS, D), jnp.float32),
    )(agg_flat, z_flat, dinv_flat, b2)


def _prep_edges(ei):
    pad = E_PAD - N_EDGES
    fill = jnp.full((pad,), N_NODES, dtype=jnp.int32)
    src = jnp.concatenate([ei[0], fill]).reshape(NUM_SUBCORES, CHUNKS_PER_SUB, CHUNK)
    dst = jnp.concatenate([ei[1], fill]).reshape(NUM_SUBCORES, CHUNKS_PER_SUB, CHUNK)
    return src, dst


def kernel(x1, edge_index1, x2, edge_index2, W1, b1, W2, b2):
    src1, dst1 = _prep_edges(edge_index1)
    src2, dst2 = _prep_edges(edge_index2)
    src_all = jnp.stack([src1, src2])
    dst_all = jnp.stack([dst1, dst2])

    xp = jnp.zeros((_ROWS, D), jnp.float32)
    xp = xp.at[0:N_NODES].set(x1).at[N_PAD:N_PAD + N_NODES].set(x2)

    ones128 = jnp.ones((CHUNK, D), jnp.float32)
    zrows = jnp.zeros((CHUNK, D), jnp.float32)
    b1r = b1.reshape(1, D)
    b2r = b2.reshape(1, D)

    xw1 = _tc_matmul(xp, W1)
    hist = _sc_degree(dst_all, ones128, zrows)
    hist_flat = hist.reshape(_ROWS, D)

    z1, dinv = _tc_layer1(hist_flat, xw1)
    agg1 = _sc_scatter_add(z1.reshape(NUM_CORES, N_PAD, D), src_all, dst_all,
                           zrows).reshape(_ROWS, D)
    z2 = _tc_layer2(agg1, z1, dinv, b1r, W2)
    agg2 = _sc_scatter_add(z2.reshape(NUM_CORES, N_PAD, D), src_all, dst_all,
                           zrows).reshape(_ROWS, D)
    out = _tc_final(agg2, z2, dinv, b2r)

    u = out[0:N_NODES]
    v = out[N_PAD:N_PAD + N_NODES]
    return (u, v)


# speedup vs baseline: 1.0158x; 1.0158x over previous
import functools

import jax
import jax.numpy as jnp
from jax import lax
from jax.experimental import pallas as pl
from jax.experimental.pallas import tpu as pltpu
from jax.experimental.pallas import tpu_sc as plsc

N_NODES = 10000
N_EDGES = 320000
D = 128

NUM_CORES = 2
NUM_SUBCORES = 16
CHUNK = 128
ROWS_PER_SUB = 640
N_PAD = NUM_SUBCORES * ROWS_PER_SUB
IDX_BLOCK = 32
IDX_BLOCKS = 5
CHUNKS_PER_SUB = IDX_BLOCK * IDX_BLOCKS
EDGES_PER_SUB = CHUNKS_PER_SUB * CHUNK
E_PAD = NUM_SUBCORES * EDGES_PER_SUB


def _sc_mesh():
    return plsc.VectorSubcoreMesh(
        core_axis_name="c", subcore_axis_name="s",
        num_cores=NUM_CORES, num_subcores=NUM_SUBCORES)


@jax.jit
def _sc_degree(dst_all, ones_hbm, zeros_hbm):
    @functools.partial(
        pl.kernel,
        out_type=jax.ShapeDtypeStruct((NUM_CORES, N_PAD, D), jnp.float32),
        mesh=_sc_mesh(),
        scratch_types=[
            pltpu.VMEM((IDX_BLOCK, CHUNK), jnp.int32),
            pltpu.VMEM((CHUNK, D), jnp.float32),
            pltpu.VMEM_SHARED((N_PAD, D), jnp.float32),
            pltpu.SemaphoreType.DMA,
        ],
    )
    def hist_kernel(dst_hbm, ones_h, zeros_h, hist_hbm, idx_v, ones_v, acc_sh, sem):
        cid = lax.axis_index("c")
        sid = lax.axis_index("s")
        for t in range(ROWS_PER_SUB // CHUNK):
            pltpu.sync_copy(
                zeros_h, acc_sh.at[pl.ds(sid * ROWS_PER_SUB + t * CHUNK, CHUNK)])
        pltpu.sync_copy(ones_h, ones_v)
        plsc.subcore_barrier()

        dg = dst_hbm.at[cid].at[sid]

        @pl.loop(0, IDX_BLOCKS)
        def _(blk):
            pltpu.sync_copy(dg.at[pl.ds(blk * IDX_BLOCK, IDX_BLOCK)], idx_v)

            @pl.loop(0, IDX_BLOCK // 2)
            def _(j):
                a0 = pltpu.async_copy(ones_v, acc_sh.at[idx_v.at[2 * j]], sem,
                                      add=True)
                a1 = pltpu.async_copy(ones_v, acc_sh.at[idx_v.at[2 * j + 1]],
                                      sem, add=True)
                a0.wait()
                a1.wait()

        plsc.subcore_barrier()
        pltpu.sync_copy(
            acc_sh.at[pl.ds(sid * ROWS_PER_SUB, ROWS_PER_SUB)],
            hist_hbm.at[cid].at[pl.ds(sid * ROWS_PER_SUB, ROWS_PER_SUB)],
        )

    return hist_kernel(dst_all, ones_hbm, zeros_hbm)


@jax.jit
def _sc_scatter_add(z_all, src_all, dst_all, zrows_hbm):
    @functools.partial(
        pl.kernel,
        out_type=jax.ShapeDtypeStruct((NUM_CORES, N_PAD, D), jnp.float32),
        mesh=_sc_mesh(),
        scratch_types=[
            pltpu.VMEM((IDX_BLOCK, CHUNK), jnp.int32),
            pltpu.VMEM((IDX_BLOCK, CHUNK), jnp.int32),
            pltpu.VMEM((CHUNK, D), jnp.float32),
            pltpu.VMEM((CHUNK, D), jnp.float32),
            pltpu.VMEM_SHARED((N_PAD, D), jnp.float32),
            pltpu.SemaphoreType.DMA,
            pltpu.SemaphoreType.DMA,
            pltpu.SemaphoreType.DMA,
            pltpu.SemaphoreType.DMA,
        ],
    )
    def scat_kernel(z_hbm, src_hbm, dst_hbm, zr_hbm, agg_hbm,
                    src_v, dst_v, rows0, rows1, acc_sh,
                    semg0, semg1, sema0, sema1):
        cid = lax.axis_index("c")
        sid = lax.axis_index("s")
        for t in range(ROWS_PER_SUB // CHUNK):
            pltpu.sync_copy(
                zr_hbm, acc_sh.at[pl.ds(sid * ROWS_PER_SUB + t * CHUNK, CHUNK)])
        plsc.subcore_barrier()

        zg = z_hbm.at[cid]
        sg = src_hbm.at[cid].at[sid]
        dg = dst_hbm.at[cid].at[sid]

        @pl.loop(0, IDX_BLOCKS)
        def _(blk):
            pltpu.sync_copy(sg.at[pl.ds(blk * IDX_BLOCK, IDX_BLOCK)], src_v)
            pltpu.sync_copy(dg.at[pl.ds(blk * IDX_BLOCK, IDX_BLOCK)], dst_v)
            pltpu.async_copy(zg.at[src_v.at[0]], rows0, semg0)

            @pl.loop(0, IDX_BLOCK // 2)
            def _(p):
                c0 = 2 * p
                c1 = c0 + 1
                pltpu.make_async_copy(zg.at[src_v.at[c0]], rows0, semg0).wait()
                pltpu.async_copy(rows0, acc_sh.at[dst_v.at[c0]], sema0,
                                 add=True)
                @pl.when(p > 0)
                def _():
                    pltpu.make_async_copy(
                        rows1, acc_sh.at[dst_v.at[c0 - 1]], sema1).wait()
                pltpu.async_copy(zg.at[src_v.at[c1]], rows1, semg1)
                pltpu.make_async_copy(zg.at[src_v.at[c1]], rows1, semg1).wait()
                pltpu.async_copy(rows1, acc_sh.at[dst_v.at[c1]], sema1,
                                 add=True)
                pltpu.make_async_copy(rows0, acc_sh.at[dst_v.at[c0]],
                                      sema0).wait()

                @pl.when(p < IDX_BLOCK // 2 - 1)
                def _():
                    pltpu.async_copy(zg.at[src_v.at[c0 + 2]], rows0, semg0)

            pltpu.make_async_copy(
                rows1, acc_sh.at[dst_v.at[IDX_BLOCK - 1]], sema1).wait()

        plsc.subcore_barrier()
        pltpu.sync_copy(
            acc_sh.at[pl.ds(sid * ROWS_PER_SUB, ROWS_PER_SUB)],
            agg_hbm.at[cid].at[pl.ds(sid * ROWS_PER_SUB, ROWS_PER_SUB)],
        )

    return scat_kernel(z_all, src_all, dst_all, zrows_hbm)


_ROWS = 2 * N_PAD
_BLK = 1024
_GRID = _ROWS // _BLK


def _row_mask(i):
    r = i * _BLK + lax.broadcasted_iota(jnp.int32, (_BLK, 1), 0)
    return (lax.rem(r, N_PAD) < N_NODES).astype(jnp.float32)


def _tc_matmul_body(x_ref, w_ref, xw_ref):
    xw_ref[...] = jnp.dot(x_ref[...], w_ref[...],
                          preferred_element_type=jnp.float32)


@jax.jit
def _tc_matmul(x_flat, w1):
    return pl.pallas_call(
        _tc_matmul_body,
        grid=(_GRID,),
        in_specs=[
            pl.BlockSpec((_BLK, D), lambda i: (i, 0)),
            pl.BlockSpec((D, D), lambda i: (0, 0)),
        ],
        out_specs=pl.BlockSpec((_BLK, D), lambda i: (i, 0)),
        out_shape=jax.ShapeDtypeStruct((_ROWS, D), jnp.float32),
    )(x_flat, w1)


def _tc_layer1_body(hist_ref, xw_ref, z_ref, dinv_ref):
    i = pl.program_id(0)
    deg = hist_ref[:, 0:1] + 1.0
    dinv = lax.rsqrt(deg) * _row_mask(i)
    z_ref[...] = dinv * xw_ref[...]
    dinv_ref[...] = jnp.broadcast_to(dinv, (_BLK, D))


@jax.jit
def _tc_layer1(hist_flat, xw_flat):
    return pl.pallas_call(
        _tc_layer1_body,
        grid=(_GRID,),
        in_specs=[
            pl.BlockSpec((_BLK, D), lambda i: (i, 0)),
            pl.BlockSpec((_BLK, D), lambda i: (i, 0)),
        ],
        out_specs=[
            pl.BlockSpec((_BLK, D), lambda i: (i, 0)),
            pl.BlockSpec((_BLK, D), lambda i: (i, 0)),
        ],
        out_shape=[
            jax.ShapeDtypeStruct((_ROWS, D), jnp.float32),
            jax.ShapeDtypeStruct((_ROWS, D), jnp.float32),
        ],
    )(hist_flat, xw_flat)


def _tc_layer2_body(agg_ref, z_ref, dinv_ref, b_ref, w_ref, z2_ref):
    dinv = dinv_ref[...]
    h = jnp.maximum(dinv * (agg_ref[...] + z_ref[...]) + b_ref[...], 0.0)
    z2_ref[...] = dinv * jnp.dot(h, w_ref[...],
                                 preferred_element_type=jnp.float32)


@jax.jit
def _tc_layer2(agg_flat, z_flat, dinv_flat, b1, w2):
    return pl.pallas_call(
        _tc_layer2_body,
        grid=(_GRID,),
        in_specs=[
            pl.BlockSpec((_BLK, D), lambda i: (i, 0)),
            pl.BlockSpec((_BLK, D), lambda i: (i, 0)),
            pl.BlockSpec((_BLK, D), lambda i: (i, 0)),
            pl.BlockSpec((1, D), lambda i: (0, 0)),
            pl.BlockSpec((D, D), lambda i: (0, 0)),
        ],
        out_specs=pl.BlockSpec((_BLK, D), lambda i: (i, 0)),
        out_shape=jax.ShapeDtypeStruct((_ROWS, D), jnp.float32),
    )(agg_flat, z_flat, dinv_flat, b1, w2)


def _tc_final_body(agg_ref, z_ref, dinv_ref, b_ref, out_ref):
    out_ref[...] = dinv_ref[...] * (agg_ref[...] + z_ref[...]) + b_ref[...]


@jax.jit
def _tc_final(agg_flat, z_flat, dinv_flat, b2):
    return pl.pallas_call(
        _tc_final_body,
        grid=(_GRID,),
        in_specs=[
            pl.BlockSpec((_BLK, D), lambda i: (i, 0)),
            pl.BlockSpec((_BLK, D), lambda i: (i, 0)),
            pl.BlockSpec((_BLK, D), lambda i: (i, 0)),
            pl.BlockSpec((1, D), lambda i: (0, 0)),
        ],
        out_specs=pl.BlockSpec((_BLK, D), lambda i: (i, 0)),
        out_shape=jax.ShapeDtypeStruct((_ROWS, D), jnp.float32),
    )(agg_flat, z_flat, dinv_flat, b2)


def _prep_edges(ei):
    pad = E_PAD - N_EDGES
    fill = jnp.full((pad,), N_NODES, dtype=jnp.int32)
    src = jnp.concatenate([ei[0], fill]).reshape(NUM_SUBCORES, CHUNKS_PER_SUB, CHUNK)
    dst = jnp.concatenate([ei[1], fill]).reshape(NUM_SUBCORES, CHUNKS_PER_SUB, CHUNK)
    return src, dst


def kernel(x1, edge_index1, x2, edge_index2, W1, b1, W2, b2):
    src1, dst1 = _prep_edges(edge_index1)
    src2, dst2 = _prep_edges(edge_index2)
    src_all = jnp.stack([src1, src2])
    dst_all = jnp.stack([dst1, dst2])

    xp = jnp.zeros((_ROWS, D), jnp.float32)
    xp = xp.at[0:N_NODES].set(x1).at[N_PAD:N_PAD + N_NODES].set(x2)

    ones128 = jnp.ones((CHUNK, D), jnp.float32)
    zrows = jnp.zeros((CHUNK, D), jnp.float32)
    b1r = b1.reshape(1, D)
    b2r = b2.reshape(1, D)

    xw1 = _tc_matmul(xp, W1)
    hist = _sc_degree(dst_all, ones128, zrows)
    hist_flat = hist.reshape(_ROWS, D)

    z1, dinv = _tc_layer1(hist_flat, xw1)
    agg1 = _sc_scatter_add(z1.reshape(NUM_CORES, N_PAD, D), src_all, dst_all,
                           zrows).reshape(_ROWS, D)
    z2 = _tc_layer2(agg1, z1, dinv, b1r, W2)
    agg2 = _sc_scatter_add(z2.reshape(NUM_CORES, N_PAD, D), src_all, dst_all,
                           zrows).reshape(_ROWS, D)
    out = _tc_final(agg2, z2, dinv, b2r)

    u = out[0:N_NODES]
    v = out[N_PAD:N_PAD + N_NODES]
    return (u, v)
